# Initial kernel scaffold; baseline (speedup 1.0000x reference)
import jax
import jax.numpy as jnp
from jax.experimental import pallas as pl


def kernel(feat, edge_index, W_self, W_neigh, b_neigh):
    raise NotImplementedError("write your pallas kernel here")


# speedup vs baseline: 3.7803x; 3.7803x over previous
import functools

import jax
import jax.numpy as jnp
from jax import lax
from jax.experimental import pallas as pl
from jax.experimental.pallas import tpu as pltpu
from jax.experimental.pallas import tpu_sc as plsc

N_NODES = 10000
N_EDGES = 320000
D = 128
DH = D // 2

NC = 2
NS = 16
EPT = N_EDGES // NS
K = 80
CH = EPT // K
RSTRIDE = 624
RPT = 640


def _sc_body(feat2_hbm, src_hbm, dst_hbm, zd_hbm, z16_hbm, ones_hbm,
             acc_out, deg_out,
             src_v, dst_v, gidx_v, rows_v, ones_v, staged_v, stage16_v,
             acc_sh, deg_sh, sem):
    cid = lax.axis_index("c")
    sid = lax.axis_index("s")
    r0 = sid * RSTRIDE
    off = cid * N_NODES

    pltpu.sync_copy(zd_hbm, staged_v)
    pltpu.sync_copy(staged_v, acc_sh.at[pl.ds(r0, RPT)])
    pltpu.sync_copy(z16_hbm, stage16_v)
    pltpu.sync_copy(stage16_v, deg_sh.at[pl.ds(r0, RPT)])
    pltpu.sync_copy(ones_hbm, ones_v)
    plsc.subcore_barrier()

    def body(j, carry):
        base = sid * EPT + j * K
        pltpu.sync_copy(src_hbm.at[pl.ds(base, K)], src_v)
        pltpu.sync_copy(dst_hbm.at[pl.ds(base, K)], dst_v)
        for i in range(K // 16):
            sl = pl.ds(i * 16, 16)
            gidx_v[sl] = src_v[sl] + off
        pltpu.async_copy(feat2_hbm.at[gidx_v], rows_v, sem).wait()
        pltpu.sync_copy(rows_v, acc_sh.at[dst_v], add=True)
        pltpu.sync_copy(ones_v, deg_sh.at[dst_v], add=True)
        return carry

    lax.fori_loop(0, CH, body, 0)
    plsc.subcore_barrier()

    out_base = off + r0
    pltpu.sync_copy(acc_sh.at[pl.ds(r0, RPT)], staged_v)
    pltpu.sync_copy(staged_v, acc_out.at[pl.ds(out_base, RPT)])
    pltpu.sync_copy(deg_sh.at[pl.ds(r0, RPT)], stage16_v)
    pltpu.sync_copy(stage16_v, deg_out.at[cid, pl.ds(r0, RPT)])


_sc_segsum = functools.partial(
    pl.kernel,
    out_type=(
        jax.ShapeDtypeStruct((NC * N_NODES, DH), jnp.float32),
        jax.ShapeDtypeStruct((NC, N_NODES, 16), jnp.float32),
    ),
    mesh=plsc.VectorSubcoreMesh(core_axis_name="c", subcore_axis_name="s",
                                num_cores=NC, num_subcores=NS),
    scratch_types=[
        pltpu.VMEM((K,), jnp.int32),
        pltpu.VMEM((K,), jnp.int32),
        pltpu.VMEM((K,), jnp.int32),
        pltpu.VMEM((K, DH), jnp.float32),
        pltpu.VMEM((K, 16), jnp.float32),
        pltpu.VMEM((RPT, DH), jnp.float32),
        pltpu.VMEM((RPT, 16), jnp.float32),
        pltpu.VMEM_SHARED((N_NODES, DH), jnp.float32),
        pltpu.VMEM_SHARED((N_NODES, 16), jnp.float32),
        pltpu.SemaphoreType.DMA,
    ],
    compiler_params=pltpu.CompilerParams(use_tc_tiling_on_sc=False),
)(_sc_body)


def _tc_body(feat_ref, acc_ref, deg_ref, ws_ref, wn_ref, b_ref, out_ref):
    s = jnp.concatenate([acc_ref[0], acc_ref[1]], axis=1)
    inv = 1.0 / jnp.maximum(deg_ref[0, :, 0:1], 1.0)
    h_neigh = s * inv
    out_ref[...] = (
        jnp.dot(feat_ref[...], ws_ref[...], preferred_element_type=jnp.float32)
        + jnp.dot(h_neigh, wn_ref[...], preferred_element_type=jnp.float32)
        + b_ref[...]
    )


_TC_ROWS = 1000


def _tc_combine(feat, acc, deg, W_self, W_neigh, b_neigh):
    grid = (N_NODES // _TC_ROWS,)
    return pl.pallas_call(
        _tc_body,
        grid=grid,
        in_specs=[
            pl.BlockSpec((_TC_ROWS, D), lambda i: (i, 0)),
            pl.BlockSpec((NC, _TC_ROWS, DH), lambda i: (0, i, 0)),
            pl.BlockSpec((NC, _TC_ROWS, 16), lambda i: (0, i, 0)),
            pl.BlockSpec((D, D), lambda i: (0, 0)),
            pl.BlockSpec((D, D), lambda i: (0, 0)),
            pl.BlockSpec((1, D), lambda i: (0, 0)),
        ],
        out_specs=pl.BlockSpec((_TC_ROWS, D), lambda i: (i, 0)),
        out_shape=jax.ShapeDtypeStruct((N_NODES, D), jnp.float32),
    )(feat, acc, deg, W_self, W_neigh, b_neigh)


def kernel(feat, edge_index, W_self, W_neigh, b_neigh):
    src = edge_index[0].astype(jnp.int32)
    dst = edge_index[1].astype(jnp.int32)
    feat2 = jnp.concatenate([feat[:, :DH], feat[:, DH:]], axis=0)
    zd = jnp.zeros((RPT, DH), jnp.float32)
    z16 = jnp.zeros((RPT, 16), jnp.float32)
    ones = jnp.ones((K, 16), jnp.float32)
    acc, deg = _sc_segsum(feat2, src, dst, zd, z16, ones)
    return _tc_combine(feat, acc.reshape(NC, N_NODES, DH), deg,
                       W_self, W_neigh, b_neigh.reshape(1, D))
